# baseline (speedup 1.0000x reference)
import jax
import jax.numpy as jnp
from jax.experimental import pallas as pl
from jax.experimental.pallas import tpu as pltpu

_R = 16


def _body(x_ref, u_ref, *refs):
    row_refs = refs[:_R]
    out_ref = refs[_R]
    acc_ref, s_ref = refs[_R + 1:]
    kdim = row_refs[0].shape[-1]
    kinv = 1.0 / kdim
    phase = pl.program_id(0)
    i = pl.program_id(1)
    ue = jnp.sum(u_ref[...])

    @pl.when(phase == 0)
    def _():
        @pl.when(i == 0)
        def _():
            acc_ref[...] = jnp.zeros_like(acc_ref)

        total = acc_ref[...]
        for j in range(_R):
            p = row_refs[j][0]
            p = p + ue * (kinv - p)
            p = jnp.clip(p, 1e-6, 1.0)
            total = total + p
        acc_ref[...] = total

    @pl.when(phase == 1)
    def _():
        @pl.when(i == 0)
        def _():
            s_ref[0] = 1.0 / jnp.sum(acc_ref[...])

        inv = s_ref[0]
        for j in range(_R):
            p = row_refs[j][0]
            p = p + ue * (kinv - p)
            p = jnp.clip(p, 1e-6, 1.0)
            out_ref[pl.ds(j, 1), :] = p * inv


def kernel(probs, x, u, t_now, t_next):
    kdim = probs.shape[0]
    b = x.shape[0]
    assert b % _R == 0
    x_i32 = jnp.asarray(x).astype(jnp.int32)
    u_vec = jnp.ravel(jnp.asarray(u)).astype(jnp.float32)
    u_pad = jnp.zeros((1, 128), jnp.float32).at[0, : u_vec.shape[0]].set(u_vec)

    probs3 = jnp.reshape(probs, (kdim, 1, kdim))

    def row_spec(j):
        return pl.BlockSpec(
            (1, 1, kdim), lambda ph, i, xr, j=j: (xr[i * _R + j], 0, 0)
        )

    grid_spec = pltpu.PrefetchScalarGridSpec(
        num_scalar_prefetch=1,
        grid=(2, b // _R),
        in_specs=[pl.BlockSpec((1, 128), lambda ph, i, xr: (0, 0))]
        + [row_spec(j) for j in range(_R)],
        out_specs=pl.BlockSpec(
            (_R, kdim), lambda ph, i, xr: (jnp.where(ph == 0, 0, i), 0)
        ),
        scratch_shapes=[
            pltpu.VMEM((1, kdim), jnp.float32),
            pltpu.SMEM((1,), jnp.float32),
        ],
    )

    return pl.pallas_call(
        _body,
        grid_spec=grid_spec,
        out_shape=jax.ShapeDtypeStruct((b, kdim), jnp.float32),
        compiler_params=pltpu.CompilerParams(
            dimension_semantics=("arbitrary", "arbitrary"),
        ),
    )(x_i32, u_pad, *([probs3] * _R))


# speedup vs baseline: 1.2292x; 1.2292x over previous
import jax
import jax.numpy as jnp
from jax.experimental import pallas as pl
from jax.experimental.pallas import tpu as pltpu

_R = 32


def _body(x_ref, u_ref, *refs):
    row_refs = refs[:_R]
    out_ref = refs[_R]
    acc_ref, s_ref = refs[_R + 1:]
    kdim = row_refs[0].shape[-1]
    kinv = 1.0 / kdim
    phase = pl.program_id(0)
    i = pl.program_id(1)
    ue = jnp.sum(u_ref[...])

    @pl.when(phase == 0)
    def _():
        @pl.when(i == 0)
        def _():
            acc_ref[...] = jnp.zeros_like(acc_ref)

        total = acc_ref[...]
        for j in range(_R):
            p = row_refs[j][0]
            p = p + ue * (kinv - p)
            p = jnp.clip(p, 1e-6, 1.0)
            total = total + p
        acc_ref[...] = total

    @pl.when(phase == 1)
    def _():
        @pl.when(i == 0)
        def _():
            s_ref[0] = 1.0 / jnp.sum(acc_ref[...])

        inv = s_ref[0]
        for j in range(_R):
            p = row_refs[j][0]
            p = p + ue * (kinv - p)
            p = jnp.clip(p, 1e-6, 1.0)
            out_ref[pl.ds(j, 1), :] = p * inv


def kernel(probs, x, u, t_now, t_next):
    kdim = probs.shape[0]
    b = x.shape[0]
    assert b % _R == 0
    x_i32 = jnp.asarray(x).astype(jnp.int32)
    u_vec = jnp.ravel(jnp.asarray(u)).astype(jnp.float32)
    u_pad = jnp.zeros((1, 128), jnp.float32).at[0, : u_vec.shape[0]].set(u_vec)

    probs3 = jnp.reshape(probs, (kdim, 1, kdim))

    def row_spec(j):
        return pl.BlockSpec(
            (1, 1, kdim), lambda ph, i, xr, j=j: (xr[i * _R + j], 0, 0)
        )

    grid_spec = pltpu.PrefetchScalarGridSpec(
        num_scalar_prefetch=1,
        grid=(2, b // _R),
        in_specs=[pl.BlockSpec((1, 128), lambda ph, i, xr: (0, 0))]
        + [row_spec(j) for j in range(_R)],
        out_specs=pl.BlockSpec(
            (_R, kdim), lambda ph, i, xr: (jnp.where(ph == 0, 0, i), 0)
        ),
        scratch_shapes=[
            pltpu.VMEM((1, kdim), jnp.float32),
            pltpu.SMEM((1,), jnp.float32),
        ],
    )

    return pl.pallas_call(
        _body,
        grid_spec=grid_spec,
        out_shape=jax.ShapeDtypeStruct((b, kdim), jnp.float32),
        compiler_params=pltpu.CompilerParams(
            dimension_semantics=("arbitrary", "arbitrary"),
        ),
    )(x_i32, u_pad, *([probs3] * _R))
